# baseline (device time: 21389 ns/iter reference)
import jax
import jax.numpy as jnp
from jax import lax
from jax.experimental import pallas as pl
from jax.experimental.pallas import tpu as pltpu

N_DEV = 4


def kernel(A, B):
    m, k = A.shape
    _, n = B.shape
    mq = m // 4
    ms = m // 8

    def body(a_ref, b_ref, out_ref, comm_q, comm_s, send_sems, recv_sems):
        my_pos = lax.axis_index("i")
        y_partner = my_pos ^ 1
        x_partner = 3 - my_pos
        yb = my_pos & 1
        xb = my_pos >> 1
        p = yb ^ xb

        def silu(z):
            return z / (1.0 + jnp.exp(-z))

        def rdma(src_off, dst_ref, size, sem, partner):
            return pltpu.make_async_remote_copy(
                src_ref=out_ref.at[pl.ds(src_off, size)],
                dst_ref=dst_ref,
                send_sem=send_sems.at[sem],
                recv_sem=recv_sems.at[sem],
                device_id=(partner,),
                device_id_type=pl.DeviceIdType.MESH,
            )

        a_keep_q = p * mq
        a_send_q = (1 - p) * mq
        a_own = a_keep_q + xb * ms
        a_send_s = a_keep_q + (1 - xb) * ms
        b_keep_q = 2 * mq + xb * mq
        b_send_q = 2 * mq + (1 - xb) * mq
        b_own = b_keep_q + yb * ms
        b_send_s = b_keep_q + (1 - yb) * ms

        barrier_sem = pltpu.get_barrier_semaphore()
        for nbr in [y_partner, x_partner]:
            pl.semaphore_signal(
                barrier_sem, inc=1,
                device_id=(nbr,), device_id_type=pl.DeviceIdType.MESH,
            )

        def dot_block(off):
            out_ref[pl.ds(off, mq), :] = jnp.dot(
                a_ref[pl.ds(off, mq), :], b_ref[:, :],
                preferred_element_type=jnp.float32)

        dot_block(a_send_q)
        pl.semaphore_wait(barrier_sem, 2)
        a1 = rdma(a_send_q, comm_q.at[0], mq, 0, y_partner)
        a1.start()
        dot_block(b_send_q)
        b1 = rdma(b_send_q, comm_q.at[1], mq, 1, x_partner)
        b1.start()
        dot_block(a_keep_q)
        dot_block(b_keep_q)

        a1.wait_recv()
        out_ref[pl.ds(a_keep_q, mq), :] += comm_q[0, :, :]
        a2 = rdma(a_send_s, comm_s.at[0], ms, 2, x_partner)
        a2.start()

        b1.wait_recv()
        out_ref[pl.ds(b_keep_q, mq), :] += comm_q[1, :, :]
        b2 = rdma(b_send_s, comm_s.at[1], ms, 3, y_partner)
        b2.start()

        a2.wait_recv()
        out_ref[pl.ds(a_own, ms), :] = silu(
            out_ref[pl.ds(a_own, ms), :] + comm_s[0, :, :])
        a3 = rdma(a_own, out_ref.at[pl.ds(a_own, ms)], ms, 4, x_partner)
        a3.start()

        b2.wait_recv()
        out_ref[pl.ds(b_own, ms), :] = silu(
            out_ref[pl.ds(b_own, ms), :] + comm_s[1, :, :])
        b3 = rdma(b_own, out_ref.at[pl.ds(b_own, ms)], ms, 5, y_partner)
        b3.start()

        a3.wait_recv()
        a4 = rdma(a_keep_q, out_ref.at[pl.ds(a_keep_q, mq)], mq, 6, y_partner)
        a4.start()

        b3.wait_recv()
        b4 = rdma(b_keep_q, out_ref.at[pl.ds(b_keep_q, mq)], mq, 7, x_partner)
        b4.start()

        a4.wait_recv()
        b4.wait_recv()

        for s in [a1, b1, a2, b2, a3, b3, a4, b4]:
            s.wait_send()

    return pl.pallas_call(
        body,
        out_shape=jax.ShapeDtypeStruct((m, n), jnp.float32),
        in_specs=[
            pl.BlockSpec(memory_space=pltpu.VMEM),
            pl.BlockSpec(memory_space=pltpu.VMEM),
        ],
        out_specs=pl.BlockSpec(memory_space=pltpu.VMEM),
        scratch_shapes=[
            pltpu.VMEM((2, mq, n), jnp.float32),
            pltpu.VMEM((2, ms, n), jnp.float32),
            pltpu.SemaphoreType.DMA((8,)),
            pltpu.SemaphoreType.DMA((8,)),
        ],
        compiler_params=pltpu.CompilerParams(collective_id=0),
    )(A, B)


# device time: 19631 ns/iter; 1.0896x vs baseline; 1.0896x over previous
import jax
import jax.numpy as jnp
from jax import lax
from jax.experimental import pallas as pl
from jax.experimental.pallas import tpu as pltpu

N_DEV = 4
NQ = 4


def kernel(A, B):
    m, k = A.shape
    _, n = B.shape
    mq = m // NQ

    def body(a_ref, b_ref, out_ref, comm_ref, send_sems, recv_sems):
        my_pos = lax.axis_index("i")
        y_partner = my_pos ^ 1
        x_partner = 3 - my_pos

        nh = NQ // 2
        partner1 = [y_partner] * nh + [x_partner] * nh
        partner2 = [x_partner] * nh + [y_partner] * nh
        order = [q for pair in zip(range(nh), range(nh, NQ)) for q in pair]

        def rdma(q, stage, partner):
            return pltpu.make_async_remote_copy(
                src_ref=out_ref.at[pl.ds(q * mq, mq)],
                dst_ref=comm_ref.at[stage, q],
                send_sem=send_sems.at[stage, q],
                recv_sem=recv_sems.at[stage, q],
                device_id=(partner,),
                device_id_type=pl.DeviceIdType.MESH,
            )

        stage1 = [None] * NQ
        barrier_sem = pltpu.get_barrier_semaphore()
        for nbr in [y_partner, x_partner]:
            pl.semaphore_signal(
                barrier_sem, inc=1,
                device_id=(nbr,), device_id_type=pl.DeviceIdType.MESH,
            )
        for idx, q in enumerate(order):
            out_ref[q * mq:(q + 1) * mq, :] = jnp.dot(
                a_ref[q * mq:(q + 1) * mq, :], b_ref[:, :],
                preferred_element_type=jnp.float32)
            if idx == 0:
                pl.semaphore_wait(barrier_sem, 2)
            s = rdma(q, 0, partner1[q])
            s.start()
            stage1[q] = s

        stage2 = [None] * NQ
        for q in order:
            stage1[q].wait_recv()
            stage1[q].wait_send()
            out_ref[q * mq:(q + 1) * mq, :] += comm_ref[0, q, :, :]
            s = rdma(q, 1, partner2[q])
            s.start()
            stage2[q] = s

        order2 = [q for pair in zip(range(nh, NQ), range(nh)) for q in pair]
        for q in order2:
            stage2[q].wait_recv()
            stage2[q].wait_send()
            z = out_ref[q * mq:(q + 1) * mq, :] + comm_ref[1, q, :, :]
            out_ref[q * mq:(q + 1) * mq, :] = z / (1.0 + jnp.exp(-z))

    return pl.pallas_call(
        body,
        out_shape=jax.ShapeDtypeStruct((m, n), jnp.float32),
        in_specs=[
            pl.BlockSpec(memory_space=pltpu.VMEM),
            pl.BlockSpec(memory_space=pltpu.VMEM),
        ],
        out_specs=pl.BlockSpec(memory_space=pltpu.VMEM),
        scratch_shapes=[
            pltpu.VMEM((2, NQ, mq, n), jnp.float32),
            pltpu.SemaphoreType.DMA((2, NQ)),
            pltpu.SemaphoreType.DMA((2, NQ)),
        ],
        compiler_params=pltpu.CompilerParams(collective_id=0),
    )(A, B)
